# baseline (speedup 1.0000x reference)
import functools

import jax
import jax.numpy as jnp
from jax import lax
from jax.experimental import pallas as pl
from jax.experimental.pallas import tpu as pltpu
from jax.experimental.pallas import tpu_sc as plsc

NC = 2
NS = 16
NW = NC * NS
IDX_CHUNK = 128
LANES = 16


def _gather_body(uidx_hbm, ridx_hbm, vidx_hbm, uhi_hbm, vhi_hbm,
                 ulo_hbm, vlo_hbm, eh_hbm, rvh_hbm, wu_hbm,
                 bs_hbm, bo_hbm,
                 u_out, v_out, ru_out, rv_out, bsg_out, bog_out,
                 uidx_v, ridx_v, vidx_v, uhi_v, vhi_v, ulo_v, vlo_v,
                 u_v, v_v, ru_v, rv_v, bsr_v, bor_v, bsg_v, bog_v, sem):
  n_chunks = uidx_v.shape[0]
  bpw = n_chunks * IDX_CHUNK
  wid = lax.axis_index("s") * NC + lax.axis_index("c")
  base = wid * bpw

  pltpu.sync_copy(uidx_hbm.at[wid], uidx_v)
  pltpu.sync_copy(ridx_hbm.at[wid], ridx_v)
  pltpu.sync_copy(vidx_hbm.at[wid], vidx_v)
  pltpu.sync_copy(uhi_hbm.at[wid], uhi_v)
  pltpu.sync_copy(vhi_hbm.at[wid], vhi_v)
  pltpu.sync_copy(ulo_hbm.at[wid], ulo_v)
  pltpu.sync_copy(vlo_hbm.at[wid], vlo_v)

  copies = []
  for c in range(n_chunks):
    rows = pl.ds(c * IDX_CHUNK, IDX_CHUNK)
    copies.append(pltpu.async_copy(eh_hbm.at[uidx_v.at[c]], u_v.at[rows], sem))
    copies.append(pltpu.async_copy(eh_hbm.at[vidx_v.at[c]], v_v.at[rows], sem))
    copies.append(pltpu.async_copy(wu_hbm.at[ridx_v.at[c]], ru_v.at[rows], sem))
    copies.append(pltpu.async_copy(rvh_hbm.at[ridx_v.at[c]], rv_v.at[rows], sem))
    copies.append(pltpu.async_copy(bs_hbm.at[uhi_v.at[c]], bsr_v.at[rows], sem))
    copies.append(pltpu.async_copy(bo_hbm.at[vhi_v.at[c]], bor_v.at[rows], sem))
  for cp in copies:
    cp.wait()

  for g in range(bpw // LANES):
    row_ids = g * LANES + lax.iota(jnp.int32, LANES)
    c = (g * LANES) // IDX_CHUNK
    o = (g * LANES) % IDX_CHUNK
    ucol = ulo_v[c, pl.ds(o, LANES)]
    vcol = vlo_v[c, pl.ds(o, LANES)]
    bsg_v[pl.ds(g * LANES, LANES)] = plsc.load_gather(bsr_v, [row_ids, ucol])
    bog_v[pl.ds(g * LANES, LANES)] = plsc.load_gather(bor_v, [row_ids, vcol])

  out_rows = pl.ds(base, bpw)
  pltpu.sync_copy(u_v, u_out.at[out_rows])
  pltpu.sync_copy(v_v, v_out.at[out_rows])
  pltpu.sync_copy(ru_v, ru_out.at[out_rows])
  pltpu.sync_copy(rv_v, rv_out.at[out_rows])
  pltpu.sync_copy(bsg_v, bsg_out.at[out_rows])
  pltpu.sync_copy(bog_v, bog_out.at[out_rows])


@jax.jit
def _sc_gather(u_idx3, r_idx3, v_idx3, u_hi3, v_hi3, u_lo3, v_lo3,
               Eh, rvh, Wu, bs16, bo16):
  nw, n_chunks, _ = u_idx3.shape
  bpw = n_chunks * IDX_CHUNK
  b = nw * bpw
  dim = Eh.shape[1]
  f32 = jnp.float32
  i32 = jnp.int32
  mesh = plsc.VectorSubcoreMesh(core_axis_name="c", subcore_axis_name="s")
  idx_t = pltpu.VMEM((n_chunks, IDX_CHUNK), i32)
  run = pl.kernel(
      _gather_body,
      mesh=mesh,
      compiler_params=pltpu.CompilerParams(use_tc_tiling_on_sc=False,
                                           needs_layout_passes=False),
      out_type=[
          jax.ShapeDtypeStruct((b, dim), f32),
          jax.ShapeDtypeStruct((b, dim), f32),
          jax.ShapeDtypeStruct((b, dim), f32),
          jax.ShapeDtypeStruct((b, dim), f32),
          jax.ShapeDtypeStruct((b,), f32),
          jax.ShapeDtypeStruct((b,), f32),
      ],
      scratch_types=[
          idx_t, idx_t, idx_t, idx_t, idx_t, idx_t, idx_t,
          pltpu.VMEM((bpw, dim), f32),
          pltpu.VMEM((bpw, dim), f32),
          pltpu.VMEM((bpw, dim), f32),
          pltpu.VMEM((bpw, dim), f32),
          pltpu.VMEM((bpw, LANES), f32),
          pltpu.VMEM((bpw, LANES), f32),
          pltpu.VMEM((bpw,), f32),
          pltpu.VMEM((bpw,), f32),
          pltpu.SemaphoreType.DMA,
      ],
  )
  return run(u_idx3, r_idx3, v_idx3, u_hi3, v_hi3, u_lo3, v_lo3,
             Eh, rvh, Wu, bs16, bo16)


def _artanh(x):
  return 0.5 * jnp.log((1 + x) / (1 - x))


def _sqnorm(x):
  return jnp.sum(x * x, axis=-1, keepdims=True)


def _norm(x):
  return jnp.sqrt(_sqnorm(x))


def _proj(t, eps=1e-5):
  nrm = _norm(t)
  msk = (nrm >= 1).astype(t.dtype)
  return t / (nrm - eps) * msk + t * (1 - msk)


def _p_sum(x, y):
  sqxnorm = jnp.clip(_sqnorm(x), 0.0, 1 - 1e-5)
  sqynorm = jnp.clip(_sqnorm(y), 0.0, 1 - 1e-5)
  dotxy = jnp.sum(x * y, axis=-1, keepdims=True)
  numerator = (1 + 2 * dotxy + sqynorm) * x + (1 - sqxnorm) * y
  denominator = 1 + 2 * dotxy + sqxnorm * sqynorm
  return numerator / denominator


def _math_body(u_ref, v_ref, ru_ref, rv_ref, bs_ref, bo_ref, out_ref):
  u = _proj(u_ref[...])
  v = _proj(v_ref[...])
  rvh_g = _proj(rv_ref[...])
  Ru = ru_ref[...]

  normu = jnp.clip(_norm(u), 1e-10, 1 - 1e-5)
  u_e = _artanh(normu) * u / normu
  u_W = u_e * Ru
  normw = jnp.clip(_norm(u_W), 1e-10, None)
  u_m = jnp.tanh(normw) * u_W / normw
  v_m = _p_sum(v, rvh_g)
  u_m = _proj(u_m)
  v_m = _proj(v_m)
  d = _p_sum(-u_m, v_m)
  nrm = jnp.clip(jnp.sqrt(jnp.sum(d * d, axis=-1)), 1e-10, 1 - 1e-5)
  sqdist = (2.0 * _artanh(nrm)) ** 2
  out_ref[...] = -sqdist + bs_ref[...][:, 0] + bo_ref[...][:, 0]


@jax.jit
def _tc_math(u, v, ru, rv, bsg, bog):
  b, dim = u.shape
  blk = 2048
  grid = (b // blk,)
  row_spec = pl.BlockSpec((blk, dim), lambda i: (i, 0))
  one_spec = pl.BlockSpec((blk, 1), lambda i: (i, 0))
  return pl.pallas_call(
      _math_body,
      grid=grid,
      in_specs=[row_spec, row_spec, row_spec, row_spec, one_spec, one_spec],
      out_specs=pl.BlockSpec((blk,), lambda i: (i,)),
      out_shape=jax.ShapeDtypeStruct((b,), jnp.float32),
  )(u, v, ru, rv, bsg, bog)


def kernel(u_idx, r_idx, v_idx, Eh, rvh, Wu, bs, bo):
  b = u_idx.shape[0]
  n_chunks = b // (NW * IDX_CHUNK)
  shape3 = (NW, n_chunks, IDX_CHUNK)
  u_idx = u_idx.astype(jnp.int32)
  r_idx = r_idx.astype(jnp.int32)
  v_idx = v_idx.astype(jnp.int32)
  u_idx3 = u_idx.reshape(shape3)
  r_idx3 = r_idx.reshape(shape3)
  v_idx3 = v_idx.reshape(shape3)
  u_hi3 = (u_idx >> 4).reshape(shape3)
  v_hi3 = (v_idx >> 4).reshape(shape3)
  u_lo3 = (u_idx & (LANES - 1)).reshape(shape3)
  v_lo3 = (v_idx & (LANES - 1)).reshape(shape3)
  bs16 = bs.reshape(-1, LANES)
  bo16 = bo.reshape(-1, LANES)
  u, v, ru, rv, bsg, bog = _sc_gather(u_idx3, r_idx3, v_idx3,
                                      u_hi3, v_hi3, u_lo3, v_lo3,
                                      Eh, rvh, Wu, bs16, bo16)
  return _tc_math(u, v, ru, rv, bsg[:, None], bog[:, None])


# speedup vs baseline: 2.0689x; 2.0689x over previous
import functools

import jax
import jax.numpy as jnp
from jax import lax
from jax.experimental import pallas as pl
from jax.experimental.pallas import tpu as pltpu
from jax.experimental.pallas import tpu_sc as plsc

NC = 2
NS = 16
NW = NC * NS
IDX_CHUNK = 128
LANES = 16
WIN = 2048


def _stream_body(eht_hbm, ents_hbm, out_hbm, ent_v, win_v, soa_v):
  n_ent = ent_v.shape[0]
  dim, n_table = eht_hbm.shape
  pad_minor = ((n_table + 127) // 128) * 128
  cb_max = (pad_minor - WIN) // 128
  wid = lax.axis_index("s") * NC + lax.axis_index("c")
  pltpu.sync_copy(ents_hbm.at[wid], ent_v)

  iota = lax.iota(jnp.int32, LANES)
  zeros = jnp.zeros((LANES,), jnp.int32)

  def group(g, cb):
    evec = ent_v[pl.ds(g * LANES, LANES)]
    for j in range(LANES):
      e = evec[j]
      trig = (e - cb * 128) >= WIN
      newcb = jnp.minimum(lax.shift_right_logical(e, 7), cb_max)
      cb = jnp.where(trig, newcb, cb)

      @pl.when(trig)
      def _():
        off = pl.multiple_of(cb * 128, 128)
        pltpu.sync_copy(eht_hbm.at[:, pl.ds(off, WIN)], win_v)

      col = zeros + (e - cb * 128)
      k = zeros + (g * LANES + j)
      lo = plsc.load_gather(win_v, [iota, col])
      hi = plsc.load_gather(win_v, [iota + LANES, col])
      plsc.store_scatter(soa_v, [iota, k], lo)
      plsc.store_scatter(soa_v, [iota + LANES, k], hi)
    return cb

  lax.fori_loop(0, n_ent // LANES, group, jnp.int32(-(2 ** 20)))
  pltpu.sync_copy(soa_v, out_hbm.at[:, pl.ds(wid * n_ent, n_ent)])


@jax.jit
def _sc_stream_gather(EhT, ents2):
  nw, n_ent = ents2.shape
  dim = EhT.shape[0]
  mesh = plsc.VectorSubcoreMesh(core_axis_name="c", subcore_axis_name="s")
  run = pl.kernel(
      _stream_body,
      mesh=mesh,
      compiler_params=pltpu.CompilerParams(use_tc_tiling_on_sc=True,
                                           needs_layout_passes=False,
                                           disable_bounds_checks=True),
      out_type=[jax.ShapeDtypeStruct((dim, nw * n_ent), jnp.float32)],
      scratch_types=[
          pltpu.VMEM((n_ent,), jnp.int32),
          pltpu.VMEM((dim, WIN), jnp.float32),
          pltpu.VMEM((dim, n_ent), jnp.float32),
      ],
  )
  return run(EhT, ents2)


def _gather_body(pu_hbm, pv_hbm, ridx_hbm, uhi_hbm, vhi_hbm,
                 ulo_hbm, vlo_hbm, uv_hbm, rvh_hbm, wu_hbm,
                 bs_hbm, bo_hbm,
                 u_out, v_out, ru_out, rv_out, bsg_out, bog_out,
                 pu_v, pv_v, ridx_v, uhi_v, vhi_v, ulo_v, vlo_v,
                 u_v, v_v, ru_v, rv_v, bsr_v, bor_v, bsg_v, bog_v, sem):
  n_chunks = pu_v.shape[0]
  bpw = n_chunks * IDX_CHUNK
  wid = lax.axis_index("s") * NC + lax.axis_index("c")
  base = wid * bpw

  pltpu.sync_copy(pu_hbm.at[wid], pu_v)
  pltpu.sync_copy(pv_hbm.at[wid], pv_v)
  pltpu.sync_copy(ridx_hbm.at[wid], ridx_v)
  pltpu.sync_copy(uhi_hbm.at[wid], uhi_v)
  pltpu.sync_copy(vhi_hbm.at[wid], vhi_v)
  pltpu.sync_copy(ulo_hbm.at[wid], ulo_v)
  pltpu.sync_copy(vlo_hbm.at[wid], vlo_v)

  copies = []
  for c in range(n_chunks):
    rows = pl.ds(c * IDX_CHUNK, IDX_CHUNK)
    copies.append(pltpu.async_copy(uv_hbm.at[pu_v.at[c]], u_v.at[rows], sem))
    copies.append(pltpu.async_copy(uv_hbm.at[pv_v.at[c]], v_v.at[rows], sem))
    copies.append(pltpu.async_copy(wu_hbm.at[ridx_v.at[c]], ru_v.at[rows], sem))
    copies.append(pltpu.async_copy(rvh_hbm.at[ridx_v.at[c]], rv_v.at[rows], sem))
    copies.append(pltpu.async_copy(bs_hbm.at[uhi_v.at[c]], bsr_v.at[rows], sem))
    copies.append(pltpu.async_copy(bo_hbm.at[vhi_v.at[c]], bor_v.at[rows], sem))
  for cp in copies:
    cp.wait()

  for g in range(bpw // LANES):
    row_ids = g * LANES + lax.iota(jnp.int32, LANES)
    c = (g * LANES) // IDX_CHUNK
    o = (g * LANES) % IDX_CHUNK
    ucol = ulo_v[c, pl.ds(o, LANES)]
    vcol = vlo_v[c, pl.ds(o, LANES)]
    bsg_v[pl.ds(g * LANES, LANES)] = plsc.load_gather(bsr_v, [row_ids, ucol])
    bog_v[pl.ds(g * LANES, LANES)] = plsc.load_gather(bor_v, [row_ids, vcol])

  out_rows = pl.ds(base, bpw)
  pltpu.sync_copy(u_v, u_out.at[out_rows])
  pltpu.sync_copy(v_v, v_out.at[out_rows])
  pltpu.sync_copy(ru_v, ru_out.at[out_rows])
  pltpu.sync_copy(rv_v, rv_out.at[out_rows])
  pltpu.sync_copy(bsg_v, bsg_out.at[out_rows])
  pltpu.sync_copy(bog_v, bog_out.at[out_rows])


@jax.jit
def _sc_gather(pu3, pv3, r_idx3, u_hi3, v_hi3, u_lo3, v_lo3,
               uv2, rvh, Wu, bs16, bo16):
  nw, n_chunks, _ = pu3.shape
  bpw = n_chunks * IDX_CHUNK
  b = nw * bpw
  dim = uv2.shape[1]
  f32 = jnp.float32
  i32 = jnp.int32
  mesh = plsc.VectorSubcoreMesh(core_axis_name="c", subcore_axis_name="s")
  idx_t = pltpu.VMEM((n_chunks, IDX_CHUNK), i32)
  run = pl.kernel(
      _gather_body,
      mesh=mesh,
      compiler_params=pltpu.CompilerParams(use_tc_tiling_on_sc=False,
                                           needs_layout_passes=False),
      out_type=[
          jax.ShapeDtypeStruct((b, dim), f32),
          jax.ShapeDtypeStruct((b, dim), f32),
          jax.ShapeDtypeStruct((b, dim), f32),
          jax.ShapeDtypeStruct((b, dim), f32),
          jax.ShapeDtypeStruct((b,), f32),
          jax.ShapeDtypeStruct((b,), f32),
      ],
      scratch_types=[
          idx_t, idx_t, idx_t, idx_t, idx_t, idx_t, idx_t,
          pltpu.VMEM((bpw, dim), f32),
          pltpu.VMEM((bpw, dim), f32),
          pltpu.VMEM((bpw, dim), f32),
          pltpu.VMEM((bpw, dim), f32),
          pltpu.VMEM((bpw, LANES), f32),
          pltpu.VMEM((bpw, LANES), f32),
          pltpu.VMEM((bpw,), f32),
          pltpu.VMEM((bpw,), f32),
          pltpu.SemaphoreType.DMA,
      ],
  )
  return run(pu3, pv3, r_idx3, u_hi3, v_hi3, u_lo3, v_lo3,
             uv2, rvh, Wu, bs16, bo16)


def _artanh(x):
  return 0.5 * jnp.log((1 + x) / (1 - x))


def _sqnorm(x):
  return jnp.sum(x * x, axis=-1, keepdims=True)


def _norm(x):
  return jnp.sqrt(_sqnorm(x))


def _proj(t, eps=1e-5):
  nrm = _norm(t)
  msk = (nrm >= 1).astype(t.dtype)
  return t / (nrm - eps) * msk + t * (1 - msk)


def _p_sum(x, y):
  sqxnorm = jnp.clip(_sqnorm(x), 0.0, 1 - 1e-5)
  sqynorm = jnp.clip(_sqnorm(y), 0.0, 1 - 1e-5)
  dotxy = jnp.sum(x * y, axis=-1, keepdims=True)
  numerator = (1 + 2 * dotxy + sqynorm) * x + (1 - sqxnorm) * y
  denominator = 1 + 2 * dotxy + sqxnorm * sqynorm
  return numerator / denominator


def _math_body(u_ref, v_ref, ru_ref, rv_ref, bs_ref, bo_ref, out_ref):
  u = _proj(u_ref[...])
  v = _proj(v_ref[...])
  rvh_g = _proj(rv_ref[...])
  Ru = ru_ref[...]

  normu = jnp.clip(_norm(u), 1e-10, 1 - 1e-5)
  u_e = _artanh(normu) * u / normu
  u_W = u_e * Ru
  normw = jnp.clip(_norm(u_W), 1e-10, None)
  u_m = jnp.tanh(normw) * u_W / normw
  v_m = _p_sum(v, rvh_g)
  u_m = _proj(u_m)
  v_m = _proj(v_m)
  d = _p_sum(-u_m, v_m)
  nrm = jnp.clip(jnp.sqrt(jnp.sum(d * d, axis=-1)), 1e-10, 1 - 1e-5)
  sqdist = (2.0 * _artanh(nrm)) ** 2
  out_ref[...] = -sqdist + bs_ref[...][:, 0] + bo_ref[...][:, 0]


@jax.jit
def _tc_math(u, v, ru, rv, bsg, bog):
  b, dim = u.shape
  blk = 2048
  grid = (b // blk,)
  row_spec = pl.BlockSpec((blk, dim), lambda i: (i, 0))
  one_spec = pl.BlockSpec((blk, 1), lambda i: (i, 0))
  return pl.pallas_call(
      _math_body,
      grid=grid,
      in_specs=[row_spec, row_spec, row_spec, row_spec, one_spec, one_spec],
      out_specs=pl.BlockSpec((blk,), lambda i: (i,)),
      out_shape=jax.ShapeDtypeStruct((b,), jnp.float32),
  )(u, v, ru, rv, bsg, bog)


def kernel(u_idx, r_idx, v_idx, Eh, rvh, Wu, bs, bo):
  b = u_idx.shape[0]
  n_chunks = b // (NW * IDX_CHUNK)
  shape3 = (NW, n_chunks, IDX_CHUNK)
  u_idx = u_idx.astype(jnp.int32)
  r_idx = r_idx.astype(jnp.int32)
  v_idx = v_idx.astype(jnp.int32)

  ent = jnp.concatenate([u_idx, v_idx])
  order = jnp.argsort(ent).astype(jnp.int32)
  ents_sorted = jnp.take(ent, order)
  inv = jnp.zeros((2 * b,), jnp.int32).at[order].set(
      jnp.arange(2 * b, dtype=jnp.int32))
  pu3 = inv[:b].reshape(shape3)
  pv3 = inv[b:].reshape(shape3)
  ents2 = ents_sorted.reshape(NW, (2 * b) // NW)

  EhT = jnp.swapaxes(Eh, 0, 1)
  uv_sorted = _sc_stream_gather(EhT, ents2)[0]
  uv2 = jnp.swapaxes(uv_sorted, 0, 1)

  r_idx3 = r_idx.reshape(shape3)
  u_hi3 = (u_idx >> 4).reshape(shape3)
  v_hi3 = (v_idx >> 4).reshape(shape3)
  u_lo3 = (u_idx & (LANES - 1)).reshape(shape3)
  v_lo3 = (v_idx & (LANES - 1)).reshape(shape3)
  bs16 = bs.reshape(-1, LANES)
  bo16 = bo.reshape(-1, LANES)
  u, v, ru, rv, bsg, bog = _sc_gather(pu3, pv3, r_idx3,
                                      u_hi3, v_hi3, u_lo3, v_lo3,
                                      uv2, rvh, Wu, bs16, bo16)
  return _tc_math(u, v, ru, rv, bsg[:, None], bog[:, None])


# speedup vs baseline: 2.2320x; 1.0789x over previous
import functools

import jax
import jax.numpy as jnp
from jax import lax
from jax.experimental import pallas as pl
from jax.experimental.pallas import tpu as pltpu
from jax.experimental.pallas import tpu_sc as plsc

NC = 2
NS = 16
NW = NC * NS
IDX_CHUNK = 128
LANES = 16
WIN = 2048


def _stream_body(eht_hbm, ents_hbm, out_hbm, ent_v, win_v, soa_v):
  n_ent = ent_v.shape[0]
  dim, n_table = eht_hbm.shape
  pad_minor = ((n_table + 127) // 128) * 128
  cb_max = (pad_minor - WIN) // 128
  wid = lax.axis_index("s") * NC + lax.axis_index("c")
  pltpu.sync_copy(ents_hbm.at[wid], ent_v)

  iota = lax.iota(jnp.int32, LANES)
  zeros = jnp.zeros((LANES,), jnp.int32)

  def group(g, cb):
    evec = ent_v[pl.ds(g * LANES, LANES)]
    for j in range(LANES):
      e = evec[j]
      trig = (e - cb * 128) >= WIN
      newcb = jnp.minimum(lax.shift_right_logical(e, 7), cb_max)
      cb = jnp.where(trig, newcb, cb)

      @pl.when(trig)
      def _():
        off = pl.multiple_of(cb * 128, 128)
        pltpu.sync_copy(eht_hbm.at[:, pl.ds(off, WIN)], win_v)

      col = zeros + (e - cb * 128)
      k = zeros + (g * LANES + j)
      lo = plsc.load_gather(win_v, [iota, col])
      hi = plsc.load_gather(win_v, [iota + LANES, col])
      plsc.store_scatter(soa_v, [iota, k], lo)
      plsc.store_scatter(soa_v, [iota + LANES, k], hi)
    return cb

  lax.fori_loop(0, n_ent // LANES, group, jnp.int32(-(2 ** 20)))
  pltpu.sync_copy(soa_v, out_hbm.at[:, pl.ds(wid * n_ent, n_ent)])


@jax.jit
def _sc_stream_gather(EhT, ents2):
  nw, n_ent = ents2.shape
  dim = EhT.shape[0]
  mesh = plsc.VectorSubcoreMesh(core_axis_name="c", subcore_axis_name="s")
  run = pl.kernel(
      _stream_body,
      mesh=mesh,
      compiler_params=pltpu.CompilerParams(use_tc_tiling_on_sc=True,
                                           needs_layout_passes=False,
                                           disable_bounds_checks=True),
      out_type=[jax.ShapeDtypeStruct((dim, nw * n_ent), jnp.float32)],
      scratch_types=[
          pltpu.VMEM((n_ent,), jnp.int32),
          pltpu.VMEM((dim, WIN), jnp.float32),
          pltpu.VMEM((dim, n_ent), jnp.float32),
      ],
  )
  return run(EhT, ents2)


def _gather_body(order_hbm, ridx_hbm, uhi_hbm, vhi_hbm,
                 ulo_hbm, vlo_hbm, uv_hbm, rvh_hbm, wu_hbm,
                 bs_hbm, bo_hbm,
                 uv_out, ru_out, rv_out, bsg_out, bog_out,
                 order_v, ridx_v, uhi_v, vhi_v, ulo_v, vlo_v,
                 uv_rows, ru_v, rv_v, bsr_v, bor_v, bsg_v, bog_v, sem):
  n_chunks = ridx_v.shape[0]
  bpw = n_chunks * IDX_CHUNK
  uv_chunks = order_v.shape[0]
  upw = uv_chunks * IDX_CHUNK
  wid = lax.axis_index("s") * NC + lax.axis_index("c")
  base = wid * bpw

  pltpu.sync_copy(order_hbm.at[wid], order_v)
  pltpu.sync_copy(ridx_hbm.at[wid], ridx_v)
  pltpu.sync_copy(uhi_hbm.at[wid], uhi_v)
  pltpu.sync_copy(vhi_hbm.at[wid], vhi_v)
  pltpu.sync_copy(ulo_hbm.at[wid], ulo_v)
  pltpu.sync_copy(vlo_hbm.at[wid], vlo_v)
  pltpu.sync_copy(uv_hbm.at[pl.ds(wid * upw, upw)], uv_rows)

  copies = []
  for c in range(uv_chunks):
    rows = pl.ds(c * IDX_CHUNK, IDX_CHUNK)
    copies.append(pltpu.async_copy(uv_rows.at[rows],
                                   uv_out.at[order_v.at[c]], sem))
  for c in range(n_chunks):
    rows = pl.ds(c * IDX_CHUNK, IDX_CHUNK)
    copies.append(pltpu.async_copy(wu_hbm.at[ridx_v.at[c]], ru_v.at[rows], sem))
    copies.append(pltpu.async_copy(rvh_hbm.at[ridx_v.at[c]], rv_v.at[rows], sem))
    copies.append(pltpu.async_copy(bs_hbm.at[uhi_v.at[c]], bsr_v.at[rows], sem))
    copies.append(pltpu.async_copy(bo_hbm.at[vhi_v.at[c]], bor_v.at[rows], sem))
  for cp in copies:
    cp.wait()

  for g in range(bpw // LANES):
    row_ids = g * LANES + lax.iota(jnp.int32, LANES)
    c = (g * LANES) // IDX_CHUNK
    o = (g * LANES) % IDX_CHUNK
    ucol = ulo_v[c, pl.ds(o, LANES)]
    vcol = vlo_v[c, pl.ds(o, LANES)]
    bsg_v[pl.ds(g * LANES, LANES)] = plsc.load_gather(bsr_v, [row_ids, ucol])
    bog_v[pl.ds(g * LANES, LANES)] = plsc.load_gather(bor_v, [row_ids, vcol])

  out_rows = pl.ds(base, bpw)
  pltpu.sync_copy(ru_v, ru_out.at[out_rows])
  pltpu.sync_copy(rv_v, rv_out.at[out_rows])
  pltpu.sync_copy(bsg_v, bsg_out.at[out_rows])
  pltpu.sync_copy(bog_v, bog_out.at[out_rows])


@jax.jit
def _sc_gather(order3, r_idx3, u_hi3, v_hi3, u_lo3, v_lo3,
               uv2, rvh, Wu, bs16, bo16):
  nw, n_chunks, _ = r_idx3.shape
  uv_chunks = order3.shape[1]
  bpw = n_chunks * IDX_CHUNK
  upw = uv_chunks * IDX_CHUNK
  b = nw * bpw
  dim = uv2.shape[1]
  f32 = jnp.float32
  i32 = jnp.int32
  mesh = plsc.VectorSubcoreMesh(core_axis_name="c", subcore_axis_name="s")
  idx_t = pltpu.VMEM((n_chunks, IDX_CHUNK), i32)
  run = pl.kernel(
      _gather_body,
      mesh=mesh,
      compiler_params=pltpu.CompilerParams(use_tc_tiling_on_sc=False,
                                           needs_layout_passes=False),
      out_type=[
          jax.ShapeDtypeStruct((2 * b, dim), f32),
          jax.ShapeDtypeStruct((b, dim), f32),
          jax.ShapeDtypeStruct((b, dim), f32),
          jax.ShapeDtypeStruct((b,), f32),
          jax.ShapeDtypeStruct((b,), f32),
      ],
      scratch_types=[
          pltpu.VMEM((uv_chunks, IDX_CHUNK), i32),
          idx_t, idx_t, idx_t, idx_t, idx_t,
          pltpu.VMEM((upw, dim), f32),
          pltpu.VMEM((bpw, dim), f32),
          pltpu.VMEM((bpw, dim), f32),
          pltpu.VMEM((bpw, LANES), f32),
          pltpu.VMEM((bpw, LANES), f32),
          pltpu.VMEM((bpw,), f32),
          pltpu.VMEM((bpw,), f32),
          pltpu.SemaphoreType.DMA,
      ],
  )
  return run(order3, r_idx3, u_hi3, v_hi3, u_lo3, v_lo3,
             uv2, rvh, Wu, bs16, bo16)


def _artanh(x):
  return 0.5 * jnp.log((1 + x) / (1 - x))


def _sqnorm(x):
  return jnp.sum(x * x, axis=-1, keepdims=True)


def _norm(x):
  return jnp.sqrt(_sqnorm(x))


def _proj(t, eps=1e-5):
  nrm = _norm(t)
  msk = (nrm >= 1).astype(t.dtype)
  return t / (nrm - eps) * msk + t * (1 - msk)


def _p_sum(x, y):
  sqxnorm = jnp.clip(_sqnorm(x), 0.0, 1 - 1e-5)
  sqynorm = jnp.clip(_sqnorm(y), 0.0, 1 - 1e-5)
  dotxy = jnp.sum(x * y, axis=-1, keepdims=True)
  numerator = (1 + 2 * dotxy + sqynorm) * x + (1 - sqxnorm) * y
  denominator = 1 + 2 * dotxy + sqxnorm * sqynorm
  return numerator / denominator


def _math_body(u_ref, v_ref, ru_ref, rv_ref, bs_ref, bo_ref, out_ref):
  u = _proj(u_ref[...])
  v = _proj(v_ref[...])
  rvh_g = _proj(rv_ref[...])
  Ru = ru_ref[...]

  normu = jnp.clip(_norm(u), 1e-10, 1 - 1e-5)
  u_e = _artanh(normu) * u / normu
  u_W = u_e * Ru
  normw = jnp.clip(_norm(u_W), 1e-10, None)
  u_m = jnp.tanh(normw) * u_W / normw
  v_m = _p_sum(v, rvh_g)
  u_m = _proj(u_m)
  v_m = _proj(v_m)
  d = _p_sum(-u_m, v_m)
  nrm = jnp.clip(jnp.sqrt(jnp.sum(d * d, axis=-1)), 1e-10, 1 - 1e-5)
  sqdist = (2.0 * _artanh(nrm)) ** 2
  out_ref[...] = -sqdist + bs_ref[...][:, 0] + bo_ref[...][:, 0]


@jax.jit
def _tc_math(u, v, ru, rv, bsg, bog):
  b, dim = u.shape
  blk = 2048
  grid = (b // blk,)
  row_spec = pl.BlockSpec((blk, dim), lambda i: (i, 0))
  one_spec = pl.BlockSpec((blk, 1), lambda i: (i, 0))
  return pl.pallas_call(
      _math_body,
      grid=grid,
      in_specs=[row_spec, row_spec, row_spec, row_spec, one_spec, one_spec],
      out_specs=pl.BlockSpec((blk,), lambda i: (i,)),
      out_shape=jax.ShapeDtypeStruct((b,), jnp.float32),
  )(u, v, ru, rv, bsg, bog)


def kernel(u_idx, r_idx, v_idx, Eh, rvh, Wu, bs, bo):
  b = u_idx.shape[0]
  n_chunks = b // (NW * IDX_CHUNK)
  shape3 = (NW, n_chunks, IDX_CHUNK)
  u_idx = u_idx.astype(jnp.int32)
  r_idx = r_idx.astype(jnp.int32)
  v_idx = v_idx.astype(jnp.int32)

  ent = jnp.concatenate([u_idx, v_idx])
  order = jnp.argsort(ent).astype(jnp.int32)
  ents_sorted = jnp.take(ent, order)
  order3 = order.reshape(NW, 2 * n_chunks, IDX_CHUNK)
  ents2 = ents_sorted.reshape(NW, (2 * b) // NW)

  EhT = jnp.swapaxes(Eh, 0, 1)
  uv_sorted = _sc_stream_gather(EhT, ents2)[0]
  uv2 = jnp.swapaxes(uv_sorted, 0, 1)

  r_idx3 = r_idx.reshape(shape3)
  u_hi3 = (u_idx >> 4).reshape(shape3)
  v_hi3 = (v_idx >> 4).reshape(shape3)
  u_lo3 = (u_idx & (LANES - 1)).reshape(shape3)
  v_lo3 = (v_idx & (LANES - 1)).reshape(shape3)
  bs16 = bs.reshape(-1, LANES)
  bo16 = bo.reshape(-1, LANES)
  uv_unperm, ru, rv, bsg, bog = _sc_gather(order3, r_idx3,
                                           u_hi3, v_hi3, u_lo3, v_lo3,
                                           uv2, rvh, Wu, bs16, bo16)
  u = uv_unperm[:b]
  v = uv_unperm[b:]
  return _tc_math(u, v, ru, rv, bsg[:, None], bog[:, None])


# speedup vs baseline: 2.6862x; 1.2035x over previous
import functools

import jax
import jax.numpy as jnp
from jax import lax
from jax.experimental import pallas as pl
from jax.experimental.pallas import tpu as pltpu
from jax.experimental.pallas import tpu_sc as plsc

NC = 2
NS = 16
NW = NC * NS
IDX_CHUNK = 128
LANES = 16
WIN = 2048


def _stream_body(eht_hbm, ents_hbm, out_hbm, ent_v, win_v, soa_v):
  n_ent = ent_v.shape[0]
  dim, n_table = eht_hbm.shape
  pad_minor = ((n_table + 127) // 128) * 128
  cb_max = (pad_minor - WIN) // 128
  wid = lax.axis_index("s") * NC + lax.axis_index("c")
  pltpu.sync_copy(ents_hbm.at[wid], ent_v)

  iota = lax.iota(jnp.int32, LANES)
  zeros = jnp.zeros((LANES,), jnp.int32)

  def group(g, cb):
    evec = ent_v[pl.ds(g * LANES, LANES)]
    for j in range(LANES):
      e = evec[j]
      trig = (e - cb * 128) >= WIN
      newcb = jnp.minimum(lax.shift_right_logical(e, 7), cb_max)
      cb = jnp.where(trig, newcb, cb)

      @pl.when(trig)
      def _():
        off = pl.multiple_of(cb * 128, 128)
        pltpu.sync_copy(eht_hbm.at[:, pl.ds(off, WIN)], win_v)

      col = zeros + (e - cb * 128)
      k = zeros + (g * LANES + j)
      lo = plsc.load_gather(win_v, [iota, col])
      hi = plsc.load_gather(win_v, [iota + LANES, col])
      plsc.store_scatter(soa_v, [iota, k], lo)
      plsc.store_scatter(soa_v, [iota + LANES, k], hi)
    return cb

  lax.fori_loop(0, n_ent // LANES, group, jnp.int32(-(2 ** 20)))
  pltpu.sync_copy(soa_v, out_hbm.at[:, pl.ds(wid * n_ent, n_ent)])


@jax.jit
def _sc_stream_gather(EhT, ents2):
  nw, n_ent = ents2.shape
  dim = EhT.shape[0]
  mesh = plsc.VectorSubcoreMesh(core_axis_name="c", subcore_axis_name="s")
  run = pl.kernel(
      _stream_body,
      mesh=mesh,
      compiler_params=pltpu.CompilerParams(use_tc_tiling_on_sc=True,
                                           needs_layout_passes=False,
                                           disable_bounds_checks=True),
      out_type=[jax.ShapeDtypeStruct((dim, nw * n_ent), jnp.float32)],
      scratch_types=[
          pltpu.VMEM((n_ent,), jnp.int32),
          pltpu.VMEM((dim, WIN), jnp.float32),
          pltpu.VMEM((dim, n_ent), jnp.float32),
      ],
  )
  return run(EhT, ents2)


def _gather_body(order_hbm, ridx_hbm, uhi_hbm, vhi_hbm,
                 ulo_hbm, vlo_hbm, uv_hbm, rvh_hbm, wu_hbm,
                 bs_hbm, bo_hbm,
                 uv_out, ru_out, rv_out, bsg_out, bog_out,
                 order_v, ridx_v, uhi_v, vhi_v, ulo_v, vlo_v,
                 uv_rows, ru_v, rv_v, bsr_v, bor_v, bsg_v, bog_v, sem):
  n_chunks = ridx_v.shape[0]
  bpw = n_chunks * IDX_CHUNK
  uv_chunks = order_v.shape[0]
  upw = uv_chunks * IDX_CHUNK
  wid = lax.axis_index("s") * NC + lax.axis_index("c")
  base = wid * bpw

  pltpu.sync_copy(order_hbm.at[wid], order_v)
  pltpu.sync_copy(ridx_hbm.at[wid], ridx_v)
  pltpu.sync_copy(uhi_hbm.at[wid], uhi_v)
  pltpu.sync_copy(vhi_hbm.at[wid], vhi_v)
  pltpu.sync_copy(ulo_hbm.at[wid], ulo_v)
  pltpu.sync_copy(vlo_hbm.at[wid], vlo_v)
  pltpu.sync_copy(uv_hbm.at[pl.ds(wid * upw, upw)], uv_rows)

  copies = []
  for c in range(uv_chunks):
    rows = pl.ds(c * IDX_CHUNK, IDX_CHUNK)
    copies.append(pltpu.async_copy(uv_rows.at[rows],
                                   uv_out.at[order_v.at[c]], sem))
  for c in range(n_chunks):
    rows = pl.ds(c * IDX_CHUNK, IDX_CHUNK)
    copies.append(pltpu.async_copy(wu_hbm.at[ridx_v.at[c]], ru_v.at[rows], sem))
    copies.append(pltpu.async_copy(rvh_hbm.at[ridx_v.at[c]], rv_v.at[rows], sem))
    copies.append(pltpu.async_copy(bs_hbm.at[uhi_v.at[c]], bsr_v.at[rows], sem))
    copies.append(pltpu.async_copy(bo_hbm.at[vhi_v.at[c]], bor_v.at[rows], sem))
  for cp in copies:
    cp.wait()

  for g in range(bpw // LANES):
    row_ids = g * LANES + lax.iota(jnp.int32, LANES)
    c = (g * LANES) // IDX_CHUNK
    o = (g * LANES) % IDX_CHUNK
    ucol = ulo_v[c, pl.ds(o, LANES)]
    vcol = vlo_v[c, pl.ds(o, LANES)]
    bsg_v[pl.ds(g * LANES, LANES)] = plsc.load_gather(bsr_v, [row_ids, ucol])
    bog_v[pl.ds(g * LANES, LANES)] = plsc.load_gather(bor_v, [row_ids, vcol])

  out_rows = pl.ds(base, bpw)
  pltpu.sync_copy(ru_v, ru_out.at[out_rows])
  pltpu.sync_copy(rv_v, rv_out.at[out_rows])
  pltpu.sync_copy(bsg_v, bsg_out.at[out_rows])
  pltpu.sync_copy(bog_v, bog_out.at[out_rows])


@jax.jit
def _sc_gather(order3, r_idx3, u_hi3, v_hi3, u_lo3, v_lo3,
               uv2, rvh, Wu, bs16, bo16):
  nw, n_chunks, _ = r_idx3.shape
  uv_chunks = order3.shape[1]
  bpw = n_chunks * IDX_CHUNK
  upw = uv_chunks * IDX_CHUNK
  b = nw * bpw
  dim = uv2.shape[1]
  f32 = jnp.float32
  i32 = jnp.int32
  mesh = plsc.VectorSubcoreMesh(core_axis_name="c", subcore_axis_name="s")
  idx_t = pltpu.VMEM((n_chunks, IDX_CHUNK), i32)
  run = pl.kernel(
      _gather_body,
      mesh=mesh,
      compiler_params=pltpu.CompilerParams(use_tc_tiling_on_sc=False,
                                           needs_layout_passes=False),
      out_type=[
          jax.ShapeDtypeStruct((2 * b, dim), f32),
          jax.ShapeDtypeStruct((b, dim), f32),
          jax.ShapeDtypeStruct((b, dim), f32),
          jax.ShapeDtypeStruct((b,), f32),
          jax.ShapeDtypeStruct((b,), f32),
      ],
      scratch_types=[
          pltpu.VMEM((uv_chunks, IDX_CHUNK), i32),
          idx_t, idx_t, idx_t, idx_t, idx_t,
          pltpu.VMEM((upw, dim), f32),
          pltpu.VMEM((bpw, dim), f32),
          pltpu.VMEM((bpw, dim), f32),
          pltpu.VMEM((bpw, LANES), f32),
          pltpu.VMEM((bpw, LANES), f32),
          pltpu.VMEM((bpw,), f32),
          pltpu.VMEM((bpw,), f32),
          pltpu.SemaphoreType.DMA,
      ],
  )
  return run(order3, r_idx3, u_hi3, v_hi3, u_lo3, v_lo3,
             uv2, rvh, Wu, bs16, bo16)


def _artanh(x):
  return 0.5 * jnp.log((1 + x) / (1 - x))


def _sqnorm(x):
  return jnp.sum(x * x, axis=0, keepdims=True)


def _norm(x):
  return jnp.sqrt(_sqnorm(x))


def _proj(t, eps=1e-5):
  nrm = _norm(t)
  msk = (nrm >= 1).astype(t.dtype)
  return t / (nrm - eps) * msk + t * (1 - msk)


def _p_sum(x, y):
  sqxnorm = jnp.clip(_sqnorm(x), 0.0, 1 - 1e-5)
  sqynorm = jnp.clip(_sqnorm(y), 0.0, 1 - 1e-5)
  dotxy = jnp.sum(x * y, axis=0, keepdims=True)
  numerator = (1 + 2 * dotxy + sqynorm) * x + (1 - sqxnorm) * y
  denominator = 1 + 2 * dotxy + sqxnorm * sqynorm
  return numerator / denominator


def _math_body(u_ref, v_ref, ru_ref, rv_ref, bs_ref, bo_ref, out_ref):
  u = _proj(u_ref[...].T)
  v = _proj(v_ref[...].T)
  rvh_g = _proj(rv_ref[...].T)
  Ru = ru_ref[...].T

  normu = jnp.clip(_norm(u), 1e-10, 1 - 1e-5)
  u_e = _artanh(normu) * u / normu
  u_W = u_e * Ru
  normw = jnp.clip(_norm(u_W), 1e-10, None)
  u_m = jnp.tanh(normw) * u_W / normw
  v_m = _p_sum(v, rvh_g)
  u_m = _proj(u_m)
  v_m = _proj(v_m)
  d = _p_sum(-u_m, v_m)
  nrm = jnp.clip(jnp.sqrt(jnp.sum(d * d, axis=0)), 1e-10, 1 - 1e-5)
  sqdist = (2.0 * _artanh(nrm)) ** 2
  out_ref[...] = -sqdist + bs_ref[...][:, 0] + bo_ref[...][:, 0]


@jax.jit
def _tc_math(u, v, ru, rv, bsg, bog):
  b, dim = u.shape
  blk = 2048
  grid = (b // blk,)
  row_spec = pl.BlockSpec((blk, dim), lambda i: (i, 0))
  one_spec = pl.BlockSpec((blk, 1), lambda i: (i, 0))
  return pl.pallas_call(
      _math_body,
      grid=grid,
      in_specs=[row_spec, row_spec, row_spec, row_spec, one_spec, one_spec],
      out_specs=pl.BlockSpec((blk,), lambda i: (i,)),
      out_shape=jax.ShapeDtypeStruct((b,), jnp.float32),
  )(u, v, ru, rv, bsg, bog)


def kernel(u_idx, r_idx, v_idx, Eh, rvh, Wu, bs, bo):
  b = u_idx.shape[0]
  n_chunks = b // (NW * IDX_CHUNK)
  shape3 = (NW, n_chunks, IDX_CHUNK)
  u_idx = u_idx.astype(jnp.int32)
  r_idx = r_idx.astype(jnp.int32)
  v_idx = v_idx.astype(jnp.int32)

  ent = jnp.concatenate([u_idx, v_idx])
  order = jnp.argsort(ent).astype(jnp.int32)
  ents_sorted = jnp.take(ent, order)
  order3 = order.reshape(NW, 2 * n_chunks, IDX_CHUNK)
  ents2 = ents_sorted.reshape(NW, (2 * b) // NW)

  EhT = jnp.swapaxes(Eh, 0, 1)
  uv_sorted = _sc_stream_gather(EhT, ents2)[0]
  uv2 = jnp.swapaxes(uv_sorted, 0, 1)

  r_idx3 = r_idx.reshape(shape3)
  u_hi3 = (u_idx >> 4).reshape(shape3)
  v_hi3 = (v_idx >> 4).reshape(shape3)
  u_lo3 = (u_idx & (LANES - 1)).reshape(shape3)
  v_lo3 = (v_idx & (LANES - 1)).reshape(shape3)
  bs16 = bs.reshape(-1, LANES)
  bo16 = bo.reshape(-1, LANES)
  uv_unperm, ru, rv, bsg, bog = _sc_gather(order3, r_idx3,
                                           u_hi3, v_hi3, u_lo3, v_lo3,
                                           uv2, rvh, Wu, bs16, bo16)
  u = uv_unperm[:b]
  v = uv_unperm[b:]
  return _tc_math(u, v, ru, rv, bsg[:, None], bog[:, None])
